# baseline (speedup 1.0000x reference)
import functools

import jax
import jax.numpy as jnp
from jax import lax
from jax.experimental import pallas as pl
from jax.experimental.pallas import tpu as pltpu
from jax.experimental.pallas import tpu_sc as plsc

MAX_SEQ_LEN = 2048
D_MODEL = 1024
BATCH = 4

_NC = 2
_NS = 16
_NW = _NC * _NS
_ROWS = MAX_SEQ_LEN // _NW


@functools.partial(
    pl.kernel,
    mesh=plsc.VectorSubcoreMesh(core_axis_name="c", subcore_axis_name="s"),
    out_type=jax.ShapeDtypeStruct((BATCH, MAX_SEQ_LEN, D_MODEL), jnp.float32),
    scratch_types=[
        pltpu.VMEM((_ROWS, D_MODEL), jnp.float32),
        pltpu.SemaphoreType.DMA,
    ],
)
def _broadcast_table(table_hbm, out_hbm, rows_v, sem):
    wid = lax.axis_index("s") * _NC + lax.axis_index("c")
    base = wid * _ROWS
    pltpu.sync_copy(table_hbm.at[pl.ds(base, _ROWS)], rows_v)
    copies = [
        pltpu.async_copy(rows_v, out_hbm.at[b].at[pl.ds(base, _ROWS)], sem)
        for b in range(BATCH)
    ]
    for c in copies:
        c.wait()


def kernel(x, emb_table):
    del x
    return _broadcast_table(emb_table)


# speedup vs baseline: 1.0011x; 1.0011x over previous
import functools

import jax
import jax.numpy as jnp
from jax import lax
from jax.experimental import pallas as pl
from jax.experimental.pallas import tpu as pltpu
from jax.experimental.pallas import tpu_sc as plsc

MAX_SEQ_LEN = 2048
D_MODEL = 1024
BATCH = 4

_NC = 2
_NS = 16
_NW = _NC * _NS
_ROWS = MAX_SEQ_LEN // _NW


@functools.partial(
    pl.kernel,
    mesh=plsc.VectorSubcoreMesh(core_axis_name="c", subcore_axis_name="s"),
    out_type=jax.ShapeDtypeStruct((BATCH, MAX_SEQ_LEN, D_MODEL), jnp.float32),
    scratch_types=[
        pltpu.VMEM((_ROWS, D_MODEL), jnp.float32),
        pltpu.SemaphoreType.DMA,
    ],
)
def _broadcast_table(table_hbm, out_hbm, rows_v, sem):
    wid = lax.axis_index("c") * _NS + lax.axis_index("s")
    base = wid * _ROWS
    pltpu.sync_copy(table_hbm.at[pl.ds(base, _ROWS)], rows_v)
    copies = [
        pltpu.async_copy(rows_v, out_hbm.at[b].at[pl.ds(base, _ROWS)], sem)
        for b in range(BATCH)
    ]
    for c in copies:
        c.wait()


def kernel(x, emb_table):
    del x
    return _broadcast_table(emb_table)


# speedup vs baseline: 1.0049x; 1.0039x over previous
import functools

import jax
import jax.numpy as jnp
from jax import lax
from jax.experimental import pallas as pl
from jax.experimental.pallas import tpu as pltpu
from jax.experimental.pallas import tpu_sc as plsc

MAX_SEQ_LEN = 2048
D_MODEL = 1024
BATCH = 4

_NC = 2
_NS = 16
_NW = _NC * _NS
_ROWS = MAX_SEQ_LEN // _NW


@functools.partial(
    pl.kernel,
    mesh=plsc.VectorSubcoreMesh(core_axis_name="c", subcore_axis_name="s"),
    out_type=jax.ShapeDtypeStruct((BATCH, MAX_SEQ_LEN, D_MODEL), jnp.float32),
    scratch_types=[
        pltpu.VMEM((_ROWS, D_MODEL), jnp.float32),
        pltpu.SemaphoreType.DMA,
        pltpu.SemaphoreType.DMA,
        pltpu.SemaphoreType.DMA,
    ],
)
def _broadcast_table(table_hbm, out_hbm, rows_v, rsem0, rsem1, wsem):
    wid = lax.axis_index("c") * _NS + lax.axis_index("s")
    base = wid * _ROWS
    half = _ROWS // 2
    r0 = pltpu.async_copy(
        table_hbm.at[pl.ds(base, half)], rows_v.at[pl.ds(0, half)], rsem0
    )
    r1 = pltpu.async_copy(
        table_hbm.at[pl.ds(base + half, half)],
        rows_v.at[pl.ds(half, half)],
        rsem1,
    )
    writes = []
    r0.wait()
    writes += [
        pltpu.async_copy(
            rows_v.at[pl.ds(0, half)],
            out_hbm.at[b].at[pl.ds(base, half)],
            wsem,
        )
        for b in range(BATCH)
    ]
    r1.wait()
    writes += [
        pltpu.async_copy(
            rows_v.at[pl.ds(half, half)],
            out_hbm.at[b].at[pl.ds(base + half, half)],
            wsem,
        )
        for b in range(BATCH)
    ]
    for w in writes:
        w.wait()


def kernel(x, emb_table):
    del x
    return _broadcast_table(emb_table)
